# Initial kernel scaffold; baseline (speedup 1.0000x reference)
import jax
import jax.numpy as jnp
from jax.experimental import pallas as pl


def kernel(x_nchw, weight, bias):
    raise NotImplementedError("write your pallas kernel here")


# speedup vs baseline: 1.0516x; 1.0516x over previous
import functools

import jax
import jax.numpy as jnp
from jax.experimental import pallas as pl
from jax.experimental.pallas import tpu as pltpu


def _conv3x3_kernel(x_ref, w_ref, b_ref, o_ref, slab_ref, *, W, pad, nb,
                    cin, neg_slope):
    P = x_ref.shape[2]

    slab_w = slab_ref.shape[1]
    slab_ref[:, :pad + 1] = jnp.zeros((3 * cin, pad + 1), slab_ref.dtype)
    slab_ref[:, pad + P - 1:] = jnp.zeros(
        (3 * cin, slab_w - (pad + P - 1)), slab_ref.dtype)

    col = jax.lax.broadcasted_iota(jnp.int32, (1, P), 1) & (W - 1)
    m_keep_l = (col != 0).astype(jnp.float32)
    m_keep_r = (col != W - 1).astype(jnp.float32)

    def body(n, carry):
        xi = x_ref[n]
        slab_ref[0:cin, pad:pad + P] = xi.astype(jnp.bfloat16)
        slab_ref[cin:2 * cin, pad + 1:pad + 1 + P] = (
            xi * m_keep_r).astype(jnp.bfloat16)
        slab_ref[2 * cin:3 * cin, pad - 1:pad - 1 + P] = (
            xi * m_keep_l).astype(jnp.bfloat16)

        acc = jnp.dot(w_ref[0], slab_ref[:, pad - W:pad - W + P],
                      preferred_element_type=jnp.float32)
        acc = acc + jnp.dot(w_ref[1], slab_ref[:, pad:pad + P],
                            preferred_element_type=jnp.float32)
        acc = acc + jnp.dot(w_ref[2], slab_ref[:, pad + W:pad + W + P],
                            preferred_element_type=jnp.float32)
        y = acc + b_ref[...]
        o_ref[n] = jnp.maximum(y, neg_slope * y)
        return carry

    jax.lax.fori_loop(0, nb, body, 0)


@jax.jit
def _forward(x_nchw, weight, bias):
    N, Cin, H, W = x_nchw.shape
    Cout = weight.shape[0]
    P = H * W
    pad = 128

    x_flat = x_nchw.reshape(N, Cin, P)
    wb = weight.astype(jnp.bfloat16)
    w3 = jnp.stack([
        jnp.concatenate([wb[:, :, d, 1], wb[:, :, d, 0], wb[:, :, d, 2]],
                        axis=1)
        for d in range(3)])
    b2 = bias.reshape(Cout, 1)

    nb = 4
    while N % nb:
        nb -= 1
    grid = (N // nb,)

    out_flat = pl.pallas_call(
        functools.partial(_conv3x3_kernel, W=W, pad=pad, nb=nb, cin=Cin,
                          neg_slope=0.2),
        out_shape=jax.ShapeDtypeStruct((N, Cout, P), x_nchw.dtype),
        grid_spec=pltpu.PrefetchScalarGridSpec(
            num_scalar_prefetch=0,
            grid=grid,
            in_specs=[
                pl.BlockSpec((nb, Cin, P), lambda i: (i, 0, 0)),
                pl.BlockSpec((3, Cout, 3 * Cin), lambda i: (0, 0, 0)),
                pl.BlockSpec((Cout, 1), lambda i: (0, 0)),
            ],
            out_specs=pl.BlockSpec((nb, Cout, P), lambda i: (i, 0, 0)),
            scratch_shapes=[pltpu.VMEM((3 * Cin, pad + P + pad),
                                       jnp.bfloat16)],
        ),
        compiler_params=pltpu.CompilerParams(
            dimension_semantics=("parallel",),
            vmem_limit_bytes=48 * 1024 * 1024,
        ),
    )(x_flat, w3, b2)

    return out_flat.reshape(N, Cout, H, W)


def kernel(x_nchw, weight, bias):
    return _forward(x_nchw, weight, bias)
